# Initial kernel scaffold; baseline (speedup 1.0000x reference)
import jax
import jax.numpy as jnp
from jax.experimental import pallas as pl


def kernel(inputs, embedding_table, dense_w, dense_b):
    raise NotImplementedError("write your pallas kernel here")


# speedup vs baseline: 2.1287x; 2.1287x over previous
import functools

import jax
import jax.numpy as jnp
from jax import lax
from jax.experimental import pallas as pl
from jax.experimental.pallas import tpu as pltpu
from jax.experimental.pallas import tpu_sc as plsc

NC, NS, L = 2, 16, 16
NW = NC * NS
B = 16384
BPW = B // NW
V = 100
VP = 112
D = 32

_mesh = plsc.VectorSubcoreMesh(core_axis_name="c", subcore_axis_name="s")


@functools.partial(
    pl.kernel,
    out_type=jax.ShapeDtypeStruct((B,), jnp.float32),
    mesh=_mesh,
    scratch_types=[
        pltpu.VMEM((BPW,), jnp.int32),
        pltpu.VMEM((D, VP), jnp.float32),
        pltpu.VMEM((D,), jnp.float32),
        pltpu.VMEM((L,), jnp.float32),
        pltpu.VMEM((VP,), jnp.float32),
        pltpu.VMEM((BPW,), jnp.float32),
    ],
    compiler_params=pltpu.CompilerParams(needs_layout_passes=False),
)
def _sc_lut_gather(idx_hbm, tableT_hbm, w_hbm, b_hbm, out_hbm,
                   idx_v, tableT_v, w_v, b_v, lut_v, out_v):
    wid = lax.axis_index("s") * NC + lax.axis_index("c")
    base = wid * BPW

    pltpu.sync_copy(idx_hbm.at[pl.ds(base, BPW)], idx_v)
    pltpu.sync_copy(tableT_hbm, tableT_v)
    pltpu.sync_copy(w_hbm, w_v)
    pltpu.sync_copy(b_hbm, b_v)

    nchunk = VP // L
    accs = [jnp.zeros((L,), jnp.float32) for _ in range(nchunk)]
    wvecs = [w_v[pl.ds(g * L, L)] for g in range(D // L)]
    for c in range(D):
        wc = wvecs[c // L][c % L]
        for k in range(nchunk):
            accs[k] = accs[k] + tableT_v[c, pl.ds(k * L, L)] * wc
    bb = b_v[pl.ds(0, L)][0]
    for k in range(nchunk):
        x = accs[k] + bb
        lut_v[pl.ds(k * L, L)] = 1.0 / (1.0 + jnp.exp(-x))

    for j in range(BPW // L):
        iv = idx_v[pl.ds(j * L, L)]
        out_v[pl.ds(j * L, L)] = plsc.load_gather(lut_v, [iv])

    pltpu.sync_copy(out_v, out_hbm.at[pl.ds(base, BPW)])


def kernel(inputs, embedding_table, dense_w, dense_b):
    idx = inputs.reshape(B).astype(jnp.int32)
    tableT = jnp.pad(embedding_table.T, ((0, 0), (0, VP - V)))
    w = dense_w.reshape(D)
    b = jnp.pad(dense_b.astype(jnp.float32), (0, L - dense_b.shape[0]))
    out = _sc_lut_gather(idx, tableT, w, b)
    return out.reshape(B, 1)


# speedup vs baseline: 2.2742x; 1.0683x over previous
import functools

import jax
import jax.numpy as jnp
from jax import lax
from jax.experimental import pallas as pl
from jax.experimental.pallas import tpu as pltpu
from jax.experimental.pallas import tpu_sc as plsc

NC, NS, L = 2, 16, 16
NW = NC * NS
B = 16384
BPW = B // NW
V = 100
VP = 112
D = 32

_mesh = plsc.VectorSubcoreMesh(core_axis_name="c", subcore_axis_name="s")


@functools.partial(
    pl.kernel,
    out_type=jax.ShapeDtypeStruct((B,), jnp.float32),
    mesh=_mesh,
    scratch_types=[
        pltpu.VMEM((BPW,), jnp.int32),
        pltpu.VMEM((D, VP), jnp.float32),
        pltpu.VMEM((D,), jnp.float32),
        pltpu.VMEM((L,), jnp.float32),
        pltpu.VMEM((VP,), jnp.float32),
        pltpu.VMEM((BPW,), jnp.float32),
        pltpu.SemaphoreType.DMA,
        pltpu.SemaphoreType.DMA,
    ],
    compiler_params=pltpu.CompilerParams(needs_layout_passes=False),
)
def _sc_lut_gather(idx_hbm, tableT_hbm, w_hbm, b_hbm, out_hbm,
                   idx_v, tableT_v, w_v, b_v, lut_v, out_v,
                   sem_idx, sem_par):
    wid = lax.axis_index("s") * NC + lax.axis_index("c")
    base = wid * BPW

    cp_idx = pltpu.make_async_copy(idx_hbm.at[pl.ds(base, BPW)], idx_v, sem_idx)
    cp_idx.start()
    cp_tab = pltpu.make_async_copy(tableT_hbm, tableT_v, sem_par)
    cp_tab.start()
    cp_w = pltpu.make_async_copy(w_hbm, w_v, sem_par)
    cp_w.start()
    cp_b = pltpu.make_async_copy(b_hbm, b_v, sem_par)
    cp_b.start()
    cp_tab.wait()
    cp_w.wait()
    cp_b.wait()

    nchunk = VP // L
    accs = [jnp.zeros((L,), jnp.float32) for _ in range(nchunk)]
    wvecs = [w_v[pl.ds(g * L, L)] for g in range(D // L)]
    for c in range(D):
        wc = wvecs[c // L][c % L]
        for k in range(nchunk):
            accs[k] = accs[k] + tableT_v[c, pl.ds(k * L, L)] * wc
    bb = b_v[pl.ds(0, L)][0]
    for k in range(nchunk):
        x = accs[k] + bb
        lut_v[pl.ds(k * L, L)] = 1.0 / (1.0 + jnp.exp(-x))

    cp_idx.wait()
    for j in range(BPW // L):
        iv = idx_v[pl.ds(j * L, L)]
        out_v[pl.ds(j * L, L)] = plsc.load_gather(lut_v, [iv])

    pltpu.sync_copy(out_v, out_hbm.at[pl.ds(base, BPW)])


def kernel(inputs, embedding_table, dense_w, dense_b):
    idx = inputs.reshape(B).astype(jnp.int32)
    tableT = jnp.pad(embedding_table.T, ((0, 0), (0, VP - V)))
    w = dense_w.reshape(D)
    b = jnp.pad(dense_b.astype(jnp.float32), (0, L - dense_b.shape[0]))
    out = _sc_lut_gather(idx, tableT, w, b)
    return out.reshape(B, 1)
